# baseline (speedup 1.0000x reference)
import jax
import jax.numpy as jnp
from jax import lax
from jax.experimental import pallas as pl
from jax.experimental.pallas import tpu as pltpu
from jax.experimental.pallas import tpu_sc as plsc

_N = 8192
_NC = 2
_NS = 16
_NW = _NC * _NS
_RPW = _N // _NW
_CR = 4
_NCHUNK = _RPW // _CR
_NBUF = 2
_PF = 1


def _sc_body(x_hbm, v_hbm, out_hbm, buf, band, vals, in_sems, out_sems, wo_sems):
    sid = lax.axis_index("s")
    wid = lax.axis_index("c") * _NS + sid
    r0 = wid * _RPW

    pltpu.make_async_copy(
        v_hbm.at[pl.ds(r0, _RPW)], vals, in_sems.at[0]
    ).start()
    pltpu.make_async_copy(
        x_hbm.at[pl.ds(r0, 128), pl.ds(r0, 128)], band.at[pl.ds(0, 128), :],
        in_sems.at[1],
    ).start()
    pltpu.make_async_copy(
        x_hbm.at[pl.ds(r0 + 128, 128), pl.ds(r0 + 128, 128)],
        band.at[pl.ds(128, 128), :],
        in_sems.at[2],
    ).start()
    pltpu.make_async_copy(v_hbm.at[pl.ds(r0, _RPW)], vals, in_sems.at[0]).wait()
    pltpu.make_async_copy(
        x_hbm.at[pl.ds(r0, 128), pl.ds(r0, 128)], band.at[pl.ds(0, 128), :],
        in_sems.at[1],
    ).wait()
    pltpu.make_async_copy(
        x_hbm.at[pl.ds(r0, 128), pl.ds(r0, 128)], band.at[pl.ds(128, 128), :],
        in_sems.at[2],
    ).wait()

    iota = lax.broadcasted_iota(jnp.int32, (16,), 0)
    for k in range(_RPW // 16):
        vals_v = vals[pl.ds(k * 16, 16)]
        rows_v = k * 16 + iota
        cols_v = (k % 8) * 16 + iota
        plsc.addupdate_scatter(band, [rows_v, cols_v], vals_v)

    def start_in(c, b):
        pltpu.make_async_copy(
            x_hbm.at[pl.ds(r0 + c * _CR, _CR), :], buf.at[sid, b], in_sems.at[b]
        ).start()

    def wait_in(b):
        pltpu.make_async_copy(
            x_hbm.at[pl.ds(r0, _CR), :], buf.at[sid, b], in_sems.at[b]
        ).wait()

    def start_out(c, b):
        pltpu.make_async_copy(
            buf.at[sid, b], out_hbm.at[pl.ds(r0 + c * _CR, _CR), :], out_sems.at[b]
        ).start()

    def wait_out(b):
        pltpu.make_async_copy(
            buf.at[sid, b], out_hbm.at[pl.ds(r0, _CR), :], out_sems.at[b]
        ).wait()

    def patch(c, b):
        w = pl.multiple_of(r0 + 128 * ((c * _CR) // 128), 128)
        pltpu.make_async_copy(
            band.at[pl.ds(c * _CR, _CR), :],
            buf.at[sid, b, :, pl.ds(w, 128)],
            wo_sems.at[b],
        ).start()

    def wait_patch(b):
        pltpu.make_async_copy(
            band.at[pl.ds(0, _CR), :], buf.at[sid, b, :, pl.ds(0, 128)],
            wo_sems.at[b],
        ).wait()

    for b in range(_PF):
        start_in(b, b)

    def outer(o, _):
        for b in range(_NBUF):
            c = o * _NBUF + b
            wait_in(b)
            patch(c, b)
            nb = (b + _PF) % _NBUF

            @pl.when(c + _PF < _NCHUNK)
            def _():
                @pl.when(c + _PF >= _NBUF)
                def _():
                    wait_out(nb)

                start_in(c + _PF, nb)

            wait_patch(b)
            start_out(c, b)

        return ()

    lax.fori_loop(0, (_NCHUNK // _NBUF) * _NBUF // _NBUF, outer, ())

    for c in range((_NCHUNK // _NBUF) * _NBUF, _NCHUNK):
        b = c % _NBUF
        wait_in(b)
        patch(c, b)
        wait_patch(b)
        start_out(c, b)

    for b in range(_NBUF):
        wait_out(b)


def kernel(x, values):
    mesh = plsc.VectorSubcoreMesh(
        core_axis_name="c", subcore_axis_name="s", num_cores=_NC, num_subcores=_NS
    )
    f = pl.kernel(
        _sc_body,
        out_type=jax.ShapeDtypeStruct((_N, _N), jnp.float32),
        mesh=mesh,
        scratch_types=[
            pltpu.MemorySpace.VMEM_SHARED((_NS, _NBUF, _CR, _N), jnp.float32),
            pltpu.VMEM((_RPW, 128), jnp.float32),
            pltpu.VMEM((_RPW,), jnp.float32),
            pltpu.SemaphoreType.DMA((_NBUF,)),
            pltpu.SemaphoreType.DMA((_NBUF,)),
            pltpu.SemaphoreType.DMA((_NBUF,)),
        ],
        compiler_params=pltpu.CompilerParams(needs_layout_passes=False),
    )
    return f(x, values)
